# baseline (speedup 1.0000x reference)
import functools

import jax
import jax.numpy as jnp
from jax import lax
from jax.experimental import pallas as pl
from jax.experimental.pallas import tpu as pltpu
from jax.experimental.pallas import tpu_sc as plsc

_ALPHA = 0.1
_LANES = 16


def _sc_partials(features, edge_index, weight):
    n, d = features.shape
    e = weight.shape[0]
    info = plsc.get_sparse_core_info()
    nc, ns = info.num_cores, info.num_subcores
    nw = nc * ns

    per_tile = e // nw
    assert per_tile * nw == e
    chunk = 16
    for c in range(128, 15, -16):
        if per_tile % c == 0:
            chunk = c
            break
    n_full = per_tile // chunk
    tail_e = per_tile - n_full * chunk
    assert tail_e % _LANES == 0

    rp = 8
    for c in range(80, 7, -8):
        if n % c == 0:
            rp = c
            break
    n_row_pieces = n // rp
    assert rp * n_row_pieces == n

    mesh = plsc.VectorSubcoreMesh(core_axis_name="c", subcore_axis_name="s")

    @functools.partial(
        pl.kernel,
        mesh=mesh,
        out_type=jax.ShapeDtypeStruct((nc, n, d), jnp.float32),
        scratch_types=[
            pltpu.VMEM_SHARED((n, d), jnp.float32),
            pltpu.VMEM((per_tile,), jnp.int32),
            pltpu.VMEM((per_tile,), jnp.int32),
            pltpu.VMEM((per_tile,), jnp.float32),
            pltpu.VMEM((chunk, d), jnp.float32),
            pltpu.VMEM((chunk, d), jnp.float32),
            pltpu.SemaphoreType.DMA,
            pltpu.SemaphoreType.DMA,
            pltpu.SemaphoreType.DMA,
        ],
    )
    def sc_kernel(feat_hbm, ei_hbm, w_hbm, out_hbm,
                  hi_sh, src_v, dst_v, w_v, rows_a, rows_b,
                  sem_a, sem_b, sem_s):
        rows_v = rows_a
        cid = lax.axis_index("c")
        sid = lax.axis_index("s")
        wid = sid * nc + cid

        edge0 = wid * per_tile
        stage = [
            (ei_hbm.at[pl.ds(e + edge0, per_tile)], src_v),
            (ei_hbm.at[pl.ds(edge0, per_tile)], dst_v),
            (w_hbm.at[pl.ds(edge0, per_tile)], w_v),
        ]
        for s_src, s_dst in stage:
            pltpu.async_copy(s_src, s_dst, sem_b)

        def zero_row(r, carry):
            for j in range(d // _LANES):
                rows_v[r, pl.ds(j * _LANES, _LANES)] = jnp.zeros(
                    (_LANES,), jnp.float32)
            return carry
        lax.fori_loop(0, chunk, zero_row, 0)

        def zero_chunk(k, carry):
            rc = sid + ns * k

            @pl.when(rc < n_row_pieces)
            def _():
                pltpu.async_copy(rows_v.at[pl.ds(0, rp)],
                                 hi_sh.at[pl.ds(rc * rp, rp)], sem_a)
            return carry
        lax.fori_loop(0, (n_row_pieces + ns - 1) // ns, zero_chunk, 0)

        def zero_drain(k, carry):
            rc = sid + ns * k

            @pl.when(rc < n_row_pieces)
            def _():
                pltpu.make_async_copy(
                    rows_v.at[pl.ds(0, rp)],
                    hi_sh.at[pl.ds(rc * rp, rp)], sem_a).wait()
            return carry
        lax.fori_loop(0, (n_row_pieces + ns - 1) // ns, zero_drain, 0)
        for s_src, s_dst in stage:
            pltpu.make_async_copy(s_src, s_dst, sem_b).wait()
        plsc.subcore_barrier()

        def start_gather(base, buf, sem, size=chunk):
            pltpu.async_copy(
                feat_hbm.at[src_v.at[pl.ds(base, size)]],
                buf.at[pl.ds(0, size)], sem)

        def wait_gather(base, buf, sem, size=chunk):
            pltpu.make_async_copy(
                feat_hbm.at[src_v.at[pl.ds(base, size)]],
                buf.at[pl.ds(0, size)], sem).wait()

        def process(base, buf, size=chunk):
            def scale_scatter_group(g, c2):
                wvec = w_v[pl.ds(base + g * _LANES, _LANES)]
                for i in range(_LANES):
                    r = g * _LANES + i
                    wspl = jnp.full((_LANES,), wvec[i], jnp.float32)
                    for j in range(d // _LANES):
                        sl = pl.ds(j * _LANES, _LANES)
                        buf[r, sl] = buf[r, sl] * wspl
                dst16 = dst_v[pl.ds(base + g * _LANES, _LANES)]
                pltpu.async_copy(buf.at[pl.ds(g * _LANES, _LANES)],
                                 hi_sh.at[dst16], sem_s, add=True)
                return c2
            for g in range(size // _LANES):
                scale_scatter_group(g, 0)

            def drain_group(g, c2):
                dst16 = dst_v[pl.ds(base + g * _LANES, _LANES)]
                pltpu.make_async_copy(buf.at[pl.ds(g * _LANES, _LANES)],
                                      hi_sh.at[dst16], sem_s).wait()
                return c2
            lax.fori_loop(0, size // _LANES, drain_group, 0)

        start_gather(0, rows_a, sem_a)
        npairs = (n_full - 1) // 2

        def pair_body(k, carry):
            b0 = 2 * k * chunk
            b1 = b0 + chunk
            start_gather(b1, rows_b, sem_b)
            wait_gather(b0, rows_a, sem_a)
            process(b0, rows_a)
            start_gather(b0 + 2 * chunk, rows_a, sem_a)
            wait_gather(b1, rows_b, sem_b)
            process(b1, rows_b)
            return carry
        lax.fori_loop(0, npairs, pair_body, 0)

        t0 = 2 * npairs * chunk
        if n_full - 2 * npairs == 2:
            start_gather(t0 + chunk, rows_b, sem_b)
        wait_gather(t0, rows_a, sem_a)
        process(t0, rows_a)
        if n_full - 2 * npairs == 2:
            wait_gather(t0 + chunk, rows_b, sem_b)
            process(t0 + chunk, rows_b)

        if tail_e:
            tb = n_full * chunk
            start_gather(tb, rows_a, sem_a, tail_e)
            wait_gather(tb, rows_a, sem_a, tail_e)
            process(tb, rows_a, tail_e)

        plsc.subcore_barrier()

        def out_chunk(k, carry):
            rc = sid + ns * k

            @pl.when(rc < n_row_pieces)
            def _():
                pltpu.async_copy(hi_sh.at[pl.ds(rc * rp, rp)],
                                 out_hbm.at[cid, pl.ds(rc * rp, rp)], sem_a)
            return carry
        lax.fori_loop(0, (n_row_pieces + ns - 1) // ns, out_chunk, 0)

        def out_drain(k, carry):
            rc = sid + ns * k

            @pl.when(rc < n_row_pieces)
            def _():
                pltpu.make_async_copy(
                    hi_sh.at[pl.ds(rc * rp, rp)],
                    out_hbm.at[cid, pl.ds(rc * rp, rp)], sem_a).wait()
            return carry
        lax.fori_loop(0, (n_row_pieces + ns - 1) // ns, out_drain, 0)

    return sc_kernel(features, edge_index.reshape(2 * e), weight)


def _combine(p0, p1, features0):
    n, d = features0.shape
    blk = 2000
    assert n % blk == 0

    def body(p0_ref, p1_ref, f0_ref, o_ref):
        hi = p0_ref[...] + p1_ref[...]
        x = jnp.float32(1.0 - _ALPHA) * hi + jnp.float32(_ALPHA) * f0_ref[...]
        o_ref[...] = jnp.maximum(x, jnp.float32(0.0))

    spec = pl.BlockSpec((blk, d), lambda i: (i, 0))
    return pl.pallas_call(
        body,
        grid=(n // blk,),
        in_specs=[spec, spec, spec],
        out_specs=spec,
        out_shape=jax.ShapeDtypeStruct((n, d), jnp.float32),
    )(p0, p1, features0)


@jax.jit
def kernel(features, features0, edge_index, edge_weight):
    partials = _sc_partials(features, edge_index, edge_weight)
    return _combine(partials[0], partials[1], features0)


# speedup vs baseline: 1.2264x; 1.2264x over previous
import functools

import jax
import jax.numpy as jnp
from jax import lax
from jax.experimental import pallas as pl
from jax.experimental.pallas import tpu as pltpu
from jax.experimental.pallas import tpu_sc as plsc

_ALPHA = 0.1
_LANES = 16


def _sc_partials(features, edge_index, weight):
    n, d = features.shape
    e = weight.shape[0]
    info = plsc.get_sparse_core_info()
    nc, ns = info.num_cores, info.num_subcores
    nw = nc * ns

    per_tile = e // nw
    assert per_tile * nw == e
    chunk = 16
    for c in range(128, 15, -16):
        if per_tile % c == 0:
            chunk = c
            break
    n_full = per_tile // chunk
    tail_e = per_tile - n_full * chunk
    assert tail_e % _LANES == 0

    rp = 8
    for c in range(80, 7, -8):
        if n % c == 0:
            rp = c
            break
    n_row_pieces = n // rp
    assert rp * n_row_pieces == n

    mesh = plsc.VectorSubcoreMesh(core_axis_name="c", subcore_axis_name="s")

    @functools.partial(
        pl.kernel,
        mesh=mesh,
        out_type=jax.ShapeDtypeStruct((nc, n, d), jnp.float32),
        scratch_types=[
            pltpu.VMEM_SHARED((n, d), jnp.float32),
            pltpu.VMEM((per_tile,), jnp.int32),
            pltpu.VMEM((per_tile,), jnp.int32),
            pltpu.VMEM((per_tile,), jnp.float32),
            pltpu.VMEM((chunk, d), jnp.float32),
            pltpu.VMEM((chunk, d), jnp.float32),
            pltpu.SemaphoreType.DMA,
            pltpu.SemaphoreType.DMA,
            pltpu.SemaphoreType.DMA,
        ],
    )
    def sc_kernel(feat_hbm, ei_hbm, w_hbm, out_hbm,
                  hi_sh, src_v, dst_v, w_v, rows_a, rows_b,
                  sem_a, sem_b, sem_s):
        rows_v = rows_a
        cid = lax.axis_index("c")
        sid = lax.axis_index("s")
        wid = sid * nc + cid

        edge0 = wid * per_tile
        stage = [
            (ei_hbm.at[pl.ds(e + edge0, per_tile)], src_v),
            (ei_hbm.at[pl.ds(edge0, per_tile)], dst_v),
            (w_hbm.at[pl.ds(edge0, per_tile)], w_v),
        ]
        for s_src, s_dst in stage:
            pltpu.async_copy(s_src, s_dst, sem_b)

        def zero_row(r, carry):
            for j in range(d // _LANES):
                rows_v[r, pl.ds(j * _LANES, _LANES)] = jnp.zeros(
                    (_LANES,), jnp.float32)
            return carry
        lax.fori_loop(0, chunk, zero_row, 0)

        def zero_chunk(k, carry):
            rc = sid + ns * k

            @pl.when(rc < n_row_pieces)
            def _():
                pltpu.async_copy(rows_v.at[pl.ds(0, rp)],
                                 hi_sh.at[pl.ds(rc * rp, rp)], sem_a)
            return carry
        lax.fori_loop(0, (n_row_pieces + ns - 1) // ns, zero_chunk, 0)

        def zero_drain(k, carry):
            rc = sid + ns * k

            @pl.when(rc < n_row_pieces)
            def _():
                pltpu.make_async_copy(
                    rows_v.at[pl.ds(0, rp)],
                    hi_sh.at[pl.ds(rc * rp, rp)], sem_a).wait()
            return carry
        lax.fori_loop(0, (n_row_pieces + ns - 1) // ns, zero_drain, 0)
        for s_src, s_dst in stage:
            pltpu.make_async_copy(s_src, s_dst, sem_b).wait()
        plsc.subcore_barrier()

        def start_gather(base, buf, sem, size=chunk):
            pltpu.async_copy(
                feat_hbm.at[src_v.at[pl.ds(base, size)]],
                buf.at[pl.ds(0, size)], sem)

        def wait_gather(base, buf, sem, size=chunk):
            pltpu.make_async_copy(
                feat_hbm.at[src_v.at[pl.ds(base, size)]],
                buf.at[pl.ds(0, size)], sem).wait()

        def process(base, buf, size=chunk):
            def scale_scatter_group(g, c2):
                wvec = w_v[pl.ds(base + g * _LANES, _LANES)]
                for i in range(_LANES):
                    r = g * _LANES + i
                    wspl = jnp.full((_LANES,), wvec[i], jnp.float32)
                    for j in range(d // _LANES):
                        sl = pl.ds(j * _LANES, _LANES)
                        buf[r, sl] = buf[r, sl] * wspl
                dst16 = dst_v[pl.ds(base + g * _LANES, _LANES)]
                pltpu.async_copy(buf.at[pl.ds(g * _LANES, _LANES)],
                                 hi_sh.at[dst16], sem_s, add=True)
                return c2
            lax.fori_loop(0, size // _LANES, scale_scatter_group, 0)

            pltpu.make_async_copy(feat_hbm.at[pl.ds(0, size)],
                                  buf.at[pl.ds(0, size)], sem_s).wait()

        start_gather(0, rows_a, sem_a)
        npairs = (n_full - 1) // 2

        def pair_body(k, carry):
            b0 = 2 * k * chunk
            b1 = b0 + chunk
            start_gather(b1, rows_b, sem_b)
            wait_gather(b0, rows_a, sem_a)
            process(b0, rows_a)
            start_gather(b0 + 2 * chunk, rows_a, sem_a)
            wait_gather(b1, rows_b, sem_b)
            process(b1, rows_b)
            return carry
        lax.fori_loop(0, npairs, pair_body, 0)

        t0 = 2 * npairs * chunk
        if n_full - 2 * npairs == 2:
            start_gather(t0 + chunk, rows_b, sem_b)
        wait_gather(t0, rows_a, sem_a)
        process(t0, rows_a)
        if n_full - 2 * npairs == 2:
            wait_gather(t0 + chunk, rows_b, sem_b)
            process(t0 + chunk, rows_b)

        if tail_e:
            tb = n_full * chunk
            start_gather(tb, rows_a, sem_a, tail_e)
            wait_gather(tb, rows_a, sem_a, tail_e)
            process(tb, rows_a, tail_e)

        plsc.subcore_barrier()

        def out_chunk(k, carry):
            rc = sid + ns * k

            @pl.when(rc < n_row_pieces)
            def _():
                pltpu.async_copy(hi_sh.at[pl.ds(rc * rp, rp)],
                                 out_hbm.at[cid, pl.ds(rc * rp, rp)], sem_a)
            return carry
        lax.fori_loop(0, (n_row_pieces + ns - 1) // ns, out_chunk, 0)

        def out_drain(k, carry):
            rc = sid + ns * k

            @pl.when(rc < n_row_pieces)
            def _():
                pltpu.make_async_copy(
                    hi_sh.at[pl.ds(rc * rp, rp)],
                    out_hbm.at[cid, pl.ds(rc * rp, rp)], sem_a).wait()
            return carry
        lax.fori_loop(0, (n_row_pieces + ns - 1) // ns, out_drain, 0)

    return sc_kernel(features, edge_index.reshape(2 * e), weight)


def _combine(p0, p1, features0):
    n, d = features0.shape
    blk = 2000
    assert n % blk == 0

    def body(p0_ref, p1_ref, f0_ref, o_ref):
        hi = p0_ref[...] + p1_ref[...]
        x = jnp.float32(1.0 - _ALPHA) * hi + jnp.float32(_ALPHA) * f0_ref[...]
        o_ref[...] = jnp.maximum(x, jnp.float32(0.0))

    spec = pl.BlockSpec((blk, d), lambda i: (i, 0))
    return pl.pallas_call(
        body,
        grid=(n // blk,),
        in_specs=[spec, spec, spec],
        out_specs=spec,
        out_shape=jax.ShapeDtypeStruct((n, d), jnp.float32),
    )(p0, p1, features0)


@jax.jit
def kernel(features, features0, edge_index, edge_weight):
    partials = _sc_partials(features, edge_index, edge_weight)
    return _combine(partials[0], partials[1], features0)
